# baseline (speedup 1.0000x reference)
import functools

import jax
import jax.numpy as jnp
from jax import lax
from jax.experimental import pallas as pl
from jax.experimental.pallas import tpu as pltpu
from jax.experimental.pallas import tpu_sc as plsc

N = 10000
E = 320000
ND = 128
ED = 16
H = 128

NC = 2
NS = 16
NW = NC * NS
CHUNK = 80
GRP = 5
GEDGE = CHUNK * GRP
NGROUP = E // GEDGE
GPW = NGROUP // NW
NPAIR = GPW // 2
HD2 = H // 2

def _sc_mesh():
    return plsc.VectorSubcoreMesh(
        core_axis_name="c", subcore_axis_name="s", num_cores=NC, num_subcores=NS)


@functools.lru_cache(maxsize=None)
def _build_sc_gather():
    HL = HD2

    @functools.partial(
        pl.kernel,
        out_type=(
            jax.ShapeDtypeStruct((E, HL), jnp.int32),
            jax.ShapeDtypeStruct((E, HL), jnp.int32),
        ),
        mesh=_sc_mesh(),
        compiler_params=pltpu.CompilerParams(use_tc_tiling_on_sc=False),
        scratch_types=[
            pltpu.VMEM((GEDGE,), jnp.int32),
            pltpu.VMEM((GEDGE,), jnp.int32),
            pltpu.VMEM((GEDGE,), jnp.int32),
            pltpu.VMEM((GEDGE,), jnp.int32),
            pltpu.VMEM((GEDGE, HL), jnp.int32),
            pltpu.VMEM((GEDGE, HL), jnp.int32),
            pltpu.VMEM((GEDGE, HL), jnp.int32),
            pltpu.VMEM((GEDGE, HL), jnp.int32),
            pltpu.SemaphoreType.DMA,
            pltpu.SemaphoreType.DMA,
            pltpu.SemaphoreType.DMA,
            pltpu.SemaphoreType.DMA,
        ],
    )
    def _sc_gather(p_hbm, q_hbm, sidx_hbm, ridx_hbm, sg_hbm, qg_hbm,
                   si0, ri0, si1, ri1, bp0, bq0, bp1, bq1,
                   semg0, semg1, semw0, semw1):
        cid = lax.axis_index("c")
        sid = lax.axis_index("s")
        wid = sid * NC + cid
        g0 = wid * GPW

        def stage(g, siv, riv, bp, bq, semg):
            pltpu.sync_copy(sidx_hbm.at[pl.ds(g * GEDGE, GEDGE)], siv)
            pltpu.sync_copy(ridx_hbm.at[pl.ds(g * GEDGE, GEDGE)], riv)
            for t in range(GRP):
                pltpu.async_copy(p_hbm.at[siv.at[pl.ds(t * CHUNK, CHUNK)]],
                                 bp.at[pl.ds(t * CHUNK, CHUNK), :], semg)
                pltpu.async_copy(q_hbm.at[riv.at[pl.ds(t * CHUNK, CHUNK)]],
                                 bq.at[pl.ds(t * CHUNK, CHUNK), :], semg)

        def drain_gather(bp, bq, semg):
            pltpu.make_async_copy(sg_hbm.at[pl.ds(0, GEDGE), :], bp, semg).wait()
            pltpu.make_async_copy(qg_hbm.at[pl.ds(0, GEDGE), :], bq, semg).wait()

        def fire_wb(g, bp, bq, semw):
            pltpu.async_copy(bp, sg_hbm.at[pl.ds(g * GEDGE, GEDGE), :], semw)
            pltpu.async_copy(bq, qg_hbm.at[pl.ds(g * GEDGE, GEDGE), :], semw)

        def wait_wb(bp, bq, semw):
            pltpu.make_async_copy(bp, sg_hbm.at[pl.ds(0, GEDGE), :], semw).wait()
            pltpu.make_async_copy(bq, qg_hbm.at[pl.ds(0, GEDGE), :], semw).wait()

        def body(it, carry):
            ga = g0 + 2 * it
            gb = ga + 1

            @pl.when(it > 0)
            def _():
                wait_wb(bp0, bq0, semw0)

            stage(ga, si0, ri0, bp0, bq0, semg0)

            @pl.when(it > 0)
            def _():
                wait_wb(bp1, bq1, semw1)

            stage(gb, si1, ri1, bp1, bq1, semg1)
            drain_gather(bp0, bq0, semg0)
            fire_wb(ga, bp0, bq0, semw0)
            drain_gather(bp1, bq1, semg1)
            fire_wb(gb, bp1, bq1, semw1)
            return carry

        lax.fori_loop(0, NPAIR, body, 0)
        ge = g0 + 2 * NPAIR
        wait_wb(bp0, bq0, semw0)
        stage(ge, si0, ri0, bp0, bq0, semg0)
        drain_gather(bp0, bq0, semg0)
        fire_wb(ge, bp0, bq0, semw0)
        wait_wb(bp0, bq0, semw0)
        wait_wb(bp1, bq1, semw1)

    return _sc_gather


E2 = E // NC
SCH = 6400
NSCH = E2 // SCH


@functools.lru_cache(maxsize=None)
def _build_sc_scatter():
    @functools.partial(
        pl.kernel,
        out_type=jax.ShapeDtypeStruct((NC, ED, 1, N), jnp.float32),
        mesh=_sc_mesh(),
        compiler_params=pltpu.CompilerParams(needs_layout_passes=False),
        scratch_types=[
            pltpu.VMEM((SCH,), jnp.int32),
            pltpu.VMEM((SCH,), jnp.float32),
            pltpu.VMEM((SCH,), jnp.int32),
            pltpu.VMEM((SCH,), jnp.float32),
            pltpu.VMEM((N,), jnp.float32),
            pltpu.SemaphoreType.DMA,
            pltpu.SemaphoreType.DMA,
        ],
    )
    def _sc_scatter(ridx_hbm, net_hbm, zeros_hbm, out_hbm,
                    idx0, val0, idx1, val1, acc, sem0, sem1):
        cid = lax.axis_index("c")
        sid = lax.axis_index("s")
        base = cid * E2
        pltpu.sync_copy(zeros_hbm, acc)

        def fire(k, idxb, valb, sem):
            pltpu.async_copy(ridx_hbm.at[pl.ds(base + k * SCH, SCH)], idxb, sem)
            pltpu.async_copy(net_hbm.at[sid, 0, pl.ds(base + k * SCH, SCH)], valb, sem)

        def drain(idxb, valb, sem):
            pltpu.make_async_copy(ridx_hbm.at[pl.ds(base, SCH)], idxb, sem).wait()
            pltpu.make_async_copy(net_hbm.at[sid, 0, pl.ds(base, SCH)], valb, sem).wait()

        def compute(idxb, valb):
            def body(i, c2):
                for t in range(8):
                    iv = idxb[pl.ds((i * 8 + t) * 16, 16)]
                    vv = valb[pl.ds((i * 8 + t) * 16, 16)]
                    plsc.addupdate_scatter(acc, [iv], vv)
                return c2

            lax.fori_loop(0, SCH // 128, body, 0)

        fire(0, idx0, val0, sem0)

        def pair(it, carry):
            ka = 2 * it
            fire(ka + 1, idx1, val1, sem1)
            drain(idx0, val0, sem0)
            compute(idx0, val0)
            fire(ka + 2, idx0, val0, sem0)
            drain(idx1, val1, sem1)
            compute(idx1, val1)
            return carry

        lax.fori_loop(0, NSCH // 2, pair, 0)
        drain(idx0, val0, sem0)
        compute(idx0, val0)
        pltpu.sync_copy(acc, out_hbm.at[cid, sid, 0])

    return _sc_scatter


def _pack_bf16_pair(lo, hi):
    def rn(x):
        b = lax.bitcast_convert_type(x, jnp.uint32)
        return (b + 0x7FFF + ((b >> 16) & 1)) >> 16

    word = rn(lo) | (rn(hi) << 16)
    return lax.bitcast_convert_type(word, jnp.int32)


def _unpack_bf16_pair(w):
    b = lax.bitcast_convert_type(w, jnp.uint32)
    lo = lax.bitcast_convert_type(b << 16, jnp.float32)
    hi = lax.bitcast_convert_type(b & jnp.uint32(0xFFFF0000), jnp.float32)
    return lo, hi


def _prep_body(x, w1s, w1r, p_out, q_out):
    xv = x[...]
    pf = jnp.dot(xv, w1s[...], preferred_element_type=jnp.float32)
    qf = jnp.dot(xv, w1r[...], preferred_element_type=jnp.float32)
    p_out[...] = _pack_bf16_pair(pf[:, :HD2], pf[:, HD2:])
    q_out[...] = _pack_bf16_pair(qf[:, :HD2], qf[:, HD2:])


def _tc_prep(nodes, w1s, w1r):
    BN = 2000
    grid = (N // BN,)
    return pl.pallas_call(
        _prep_body,
        grid=grid,
        in_specs=[
            pl.BlockSpec((BN, ND), lambda i: (i, 0)),
            pl.BlockSpec((ND, H), lambda i: (0, 0)),
            pl.BlockSpec((ND, H), lambda i: (0, 0)),
        ],
        out_specs=[
            pl.BlockSpec((BN, HD2), lambda i: (i, 0)),
            pl.BlockSpec((BN, HD2), lambda i: (i, 0)),
        ],
        out_shape=[
            jax.ShapeDtypeStruct((N, HD2), jnp.int32),
            jax.ShapeDtypeStruct((N, HD2), jnp.int32),
        ],
    )(nodes, w1s, w1r)


def _edge_body(sg, qg, e, w1e, b1, w2, b2, b2c, eo_out, net_out):
    ev = e[...]
    sg_lo, sg_hi = _unpack_bf16_pair(sg[...])
    qg_lo, qg_hi = _unpack_bf16_pair(qg[...])
    t = jnp.dot(ev, w1e[...], preferred_element_type=jnp.float32) + b1[...]
    w2v = w2[...]
    h_lo = jax.nn.gelu(sg_lo + qg_lo + t[:, :HD2])
    h_hi = jax.nn.gelu(sg_hi + qg_hi + t[:, HD2:])
    ne = (jnp.dot(h_lo, w2v[:HD2], preferred_element_type=jnp.float32)
          + jnp.dot(h_hi, w2v[HD2:], preferred_element_type=jnp.float32)
          + b2[...])
    eo_out[...] = ev + ne
    net = (lax.dot_general(w2v[:HD2], h_lo, (((0,), (1,)), ((), ())),
                           preferred_element_type=jnp.float32)
           + lax.dot_general(w2v[HD2:], h_hi, (((0,), (1,)), ((), ())),
                             preferred_element_type=jnp.float32))
    net_out[...] = (net + b2c[...]).reshape(ED, 1, net.shape[1])


def _tc_edge(sg, qg, edges, w1e, b1, w2, b2, b2c):
    BE = 2560
    grid = (E // BE,)
    return pl.pallas_call(
        _edge_body,
        grid=grid,
        in_specs=[
            pl.BlockSpec((BE, HD2), lambda i: (i, 0)),
            pl.BlockSpec((BE, HD2), lambda i: (i, 0)),
            pl.BlockSpec((BE, ED), lambda i: (i, 0)),
            pl.BlockSpec((ED, H), lambda i: (0, 0)),
            pl.BlockSpec((1, H), lambda i: (0, 0)),
            pl.BlockSpec((H, ED), lambda i: (0, 0)),
            pl.BlockSpec((1, ED), lambda i: (0, 0)),
            pl.BlockSpec((ED, 1), lambda i: (0, 0)),
        ],
        out_specs=[
            pl.BlockSpec((BE, ED), lambda i: (i, 0)),
            pl.BlockSpec((ED, 1, BE), lambda i: (0, 0, i)),
        ],
        out_shape=[
            jax.ShapeDtypeStruct((E, ED), jnp.float32),
            jax.ShapeDtypeStruct((ED, 1, E), jnp.float32),
        ],
    )(sg, qg, edges, w1e, b1, w2, b2, b2c)


def _node_body(x, p0, p1, w1a, w1b, b1, w2, b2, out):
    xv = x[...]
    aggt = p0[...] + p1[...]
    pre = (jnp.dot(xv, w1a[...], preferred_element_type=jnp.float32)
           + lax.dot_general(aggt, w1b[...], (((0,), (0,)), ((), ())),
                             preferred_element_type=jnp.float32)
           + b1[...])
    hn = jax.nn.gelu(pre)
    out[...] = xv + jnp.dot(hn, w2[...], preferred_element_type=jnp.float32) + b2[...]


def _tc_node(nodes, p0, p1, w1a, w1b, b1, w2, b2):
    return pl.pallas_call(
        _node_body,
        grid=(1,),
        in_specs=[
            pl.BlockSpec((N, ND), lambda i: (0, 0)),
            pl.BlockSpec((ED, N), lambda i: (0, 0)),
            pl.BlockSpec((ED, N), lambda i: (0, 0)),
            pl.BlockSpec((ND, H), lambda i: (0, 0)),
            pl.BlockSpec((ED, H), lambda i: (0, 0)),
            pl.BlockSpec((1, H), lambda i: (0, 0)),
            pl.BlockSpec((H, ND), lambda i: (0, 0)),
            pl.BlockSpec((1, ND), lambda i: (0, 0)),
        ],
        out_specs=pl.BlockSpec((N, ND), lambda i: (0, 0)),
        out_shape=jax.ShapeDtypeStruct((N, ND), jnp.float32),
    )(nodes, p0, p1, w1a, w1b, b1, w2, b2)


def kernel(nodes, edges, receivers, senders,
           edge_W1, edge_b1, edge_W2, edge_b2,
           node_W1, node_b1, node_W2, node_b2):
    w1s = edge_W1[:ND]
    w1r = edge_W1[ND:2 * ND]
    w1e = edge_W1[2 * ND:]
    p, q = _tc_prep(nodes, w1s, w1r)

    sg, qg = _build_sc_gather()(p, q, senders, receivers)

    edges_out, ne_t = _tc_edge(
        sg, qg, edges, w1e, edge_b1.reshape(1, H), edge_W2,
        edge_b2.reshape(1, ED), edge_b2.reshape(ED, 1))

    zeros_acc = jnp.zeros((N,), jnp.float32)
    parts = _build_sc_scatter()(receivers, ne_t, zeros_acc)

    nodes_out = _tc_node(
        nodes, parts[0, :, 0, :], parts[1, :, 0, :],
        node_W1[:ND], node_W1[ND:], node_b1.reshape(1, H),
        node_W2, node_b2.reshape(1, ND))
    return nodes_out, edges_out


# speedup vs baseline: 1.0027x; 1.0027x over previous
import functools

import jax
import jax.numpy as jnp
from jax import lax
from jax.experimental import pallas as pl
from jax.experimental.pallas import tpu as pltpu
from jax.experimental.pallas import tpu_sc as plsc

N = 10000
E = 320000
ND = 128
ED = 16
H = 128

NC = 2
NS = 16
NW = NC * NS
CHUNK = 80
GRP = 5
GEDGE = CHUNK * GRP
NGROUP = E // GEDGE
GPW = NGROUP // NW
NPAIR = GPW // 2
HD2 = H // 2

def _sc_mesh():
    return plsc.VectorSubcoreMesh(
        core_axis_name="c", subcore_axis_name="s", num_cores=NC, num_subcores=NS)


@functools.lru_cache(maxsize=None)
def _build_sc_gather():
    HL = HD2

    @functools.partial(
        pl.kernel,
        out_type=jax.ShapeDtypeStruct((E, HL), jnp.int32),
        mesh=_sc_mesh(),
        compiler_params=pltpu.CompilerParams(use_tc_tiling_on_sc=False),
        scratch_types=[
            pltpu.VMEM((GEDGE,), jnp.int32),
            pltpu.VMEM((GEDGE,), jnp.int32),
            pltpu.VMEM((GEDGE, HL), jnp.int32),
            pltpu.VMEM((GEDGE, HL), jnp.int32),
            pltpu.SemaphoreType.DMA,
            pltpu.SemaphoreType.DMA,
            pltpu.SemaphoreType.DMA,
            pltpu.SemaphoreType.DMA,
        ],
    )
    def _sc_gather(tab_hbm, idx_hbm, out_hbm,
                   i0, i1, b0, b1, semg0, semg1, semw0, semw1):
        cid = lax.axis_index("c")
        sid = lax.axis_index("s")
        wid = sid * NC + cid
        g0 = wid * GPW

        def stage(g, iv, b, semg):
            pltpu.sync_copy(idx_hbm.at[pl.ds(g * GEDGE, GEDGE)], iv)
            for t in range(GRP):
                pltpu.async_copy(tab_hbm.at[iv.at[pl.ds(t * CHUNK, CHUNK)]],
                                 b.at[pl.ds(t * CHUNK, CHUNK), :], semg)

        def drain_gather(b, semg):
            pltpu.make_async_copy(out_hbm.at[pl.ds(0, GEDGE), :], b, semg).wait()

        def fire_wb(g, b, semw):
            pltpu.async_copy(b, out_hbm.at[pl.ds(g * GEDGE, GEDGE), :], semw)

        def wait_wb(b, semw):
            pltpu.make_async_copy(b, out_hbm.at[pl.ds(0, GEDGE), :], semw).wait()

        def body(it, carry):
            ga = g0 + 2 * it
            gb = ga + 1

            @pl.when(it > 0)
            def _():
                wait_wb(b0, semw0)

            stage(ga, i0, b0, semg0)

            @pl.when(it > 0)
            def _():
                wait_wb(b1, semw1)

            stage(gb, i1, b1, semg1)
            drain_gather(b0, semg0)
            fire_wb(ga, b0, semw0)
            drain_gather(b1, semg1)
            fire_wb(gb, b1, semw1)
            return carry

        lax.fori_loop(0, NPAIR, body, 0)
        ge = g0 + 2 * NPAIR
        wait_wb(b0, semw0)
        stage(ge, i0, b0, semg0)
        drain_gather(b0, semg0)
        fire_wb(ge, b0, semw0)
        wait_wb(b0, semw0)
        wait_wb(b1, semw1)

    return _sc_gather


E2 = E // NC
SCH = 6400
NSCH = E2 // SCH


@functools.lru_cache(maxsize=None)
def _build_sc_scatter():
    @functools.partial(
        pl.kernel,
        out_type=jax.ShapeDtypeStruct((NC, ED, 1, N), jnp.float32),
        mesh=_sc_mesh(),
        compiler_params=pltpu.CompilerParams(needs_layout_passes=False),
        scratch_types=[
            pltpu.VMEM((SCH,), jnp.int32),
            pltpu.VMEM((SCH,), jnp.float32),
            pltpu.VMEM((SCH,), jnp.int32),
            pltpu.VMEM((SCH,), jnp.float32),
            pltpu.VMEM((N,), jnp.float32),
            pltpu.SemaphoreType.DMA,
            pltpu.SemaphoreType.DMA,
        ],
    )
    def _sc_scatter(ridx_hbm, net_hbm, zeros_hbm, out_hbm,
                    idx0, val0, idx1, val1, acc, sem0, sem1):
        cid = lax.axis_index("c")
        sid = lax.axis_index("s")
        base = cid * E2
        pltpu.sync_copy(zeros_hbm, acc)

        def fire(k, idxb, valb, sem):
            pltpu.async_copy(ridx_hbm.at[pl.ds(base + k * SCH, SCH)], idxb, sem)
            pltpu.async_copy(net_hbm.at[sid, 0, pl.ds(base + k * SCH, SCH)], valb, sem)

        def drain(idxb, valb, sem):
            pltpu.make_async_copy(ridx_hbm.at[pl.ds(base, SCH)], idxb, sem).wait()
            pltpu.make_async_copy(net_hbm.at[sid, 0, pl.ds(base, SCH)], valb, sem).wait()

        def compute(idxb, valb):
            def body(i, c2):
                for t in range(8):
                    iv = idxb[pl.ds((i * 8 + t) * 16, 16)]
                    vv = valb[pl.ds((i * 8 + t) * 16, 16)]
                    plsc.addupdate_scatter(acc, [iv], vv)
                return c2

            lax.fori_loop(0, SCH // 128, body, 0)

        fire(0, idx0, val0, sem0)

        def pair(it, carry):
            ka = 2 * it
            fire(ka + 1, idx1, val1, sem1)
            drain(idx0, val0, sem0)
            compute(idx0, val0)
            fire(ka + 2, idx0, val0, sem0)
            drain(idx1, val1, sem1)
            compute(idx1, val1)
            return carry

        lax.fori_loop(0, NSCH // 2, pair, 0)
        drain(idx0, val0, sem0)
        compute(idx0, val0)
        pltpu.sync_copy(acc, out_hbm.at[cid, sid, 0])

    return _sc_scatter


def _pack_bf16_pair(lo, hi):
    def rn(x):
        b = lax.bitcast_convert_type(x, jnp.uint32)
        return (b + 0x7FFF + ((b >> 16) & 1)) >> 16

    word = rn(lo) | (rn(hi) << 16)
    return lax.bitcast_convert_type(word, jnp.int32)


def _unpack_bf16_pair(w):
    b = lax.bitcast_convert_type(w, jnp.uint32)
    lo = lax.bitcast_convert_type(b << 16, jnp.float32)
    hi = lax.bitcast_convert_type(b & jnp.uint32(0xFFFF0000), jnp.float32)
    return lo, hi


def _prep_body(x, w1s, w1r, p_out, q_out):
    xv = x[...]
    pf = jnp.dot(xv, w1s[...], preferred_element_type=jnp.float32)
    qf = jnp.dot(xv, w1r[...], preferred_element_type=jnp.float32)
    p_out[...] = _pack_bf16_pair(pf[:, :HD2], pf[:, HD2:])
    q_out[...] = _pack_bf16_pair(qf[:, :HD2], qf[:, HD2:])


def _tc_prep(nodes, w1s, w1r):
    BN = 2000
    grid = (N // BN,)
    return pl.pallas_call(
        _prep_body,
        grid=grid,
        in_specs=[
            pl.BlockSpec((BN, ND), lambda i: (i, 0)),
            pl.BlockSpec((ND, H), lambda i: (0, 0)),
            pl.BlockSpec((ND, H), lambda i: (0, 0)),
        ],
        out_specs=[
            pl.BlockSpec((BN, HD2), lambda i: (i, 0)),
            pl.BlockSpec((BN, HD2), lambda i: (i, 0)),
        ],
        out_shape=[
            jax.ShapeDtypeStruct((N, HD2), jnp.int32),
            jax.ShapeDtypeStruct((N, HD2), jnp.int32),
        ],
    )(nodes, w1s, w1r)


def _edge_body(sg, qg, e, w1e, b1, w2, b2, b2c, eo_out, net_out):
    ev = e[...]
    sg_lo, sg_hi = _unpack_bf16_pair(sg[...])
    qg_lo, qg_hi = _unpack_bf16_pair(qg[...])
    t = jnp.dot(ev, w1e[...], preferred_element_type=jnp.float32) + b1[...]
    w2v = w2[...]
    h_lo = jax.nn.gelu(sg_lo + qg_lo + t[:, :HD2])
    h_hi = jax.nn.gelu(sg_hi + qg_hi + t[:, HD2:])
    ne = (jnp.dot(h_lo, w2v[:HD2], preferred_element_type=jnp.float32)
          + jnp.dot(h_hi, w2v[HD2:], preferred_element_type=jnp.float32)
          + b2[...])
    eo_out[...] = ev + ne
    net = (lax.dot_general(w2v[:HD2], h_lo, (((0,), (1,)), ((), ())),
                           preferred_element_type=jnp.float32)
           + lax.dot_general(w2v[HD2:], h_hi, (((0,), (1,)), ((), ())),
                             preferred_element_type=jnp.float32))
    net_out[...] = (net + b2c[...]).reshape(ED, 1, net.shape[1])


def _tc_edge(sg, qg, edges, w1e, b1, w2, b2, b2c):
    BE = 2560
    grid = (E // BE,)
    return pl.pallas_call(
        _edge_body,
        grid=grid,
        in_specs=[
            pl.BlockSpec((BE, HD2), lambda i: (i, 0)),
            pl.BlockSpec((BE, HD2), lambda i: (i, 0)),
            pl.BlockSpec((BE, ED), lambda i: (i, 0)),
            pl.BlockSpec((ED, H), lambda i: (0, 0)),
            pl.BlockSpec((1, H), lambda i: (0, 0)),
            pl.BlockSpec((H, ED), lambda i: (0, 0)),
            pl.BlockSpec((1, ED), lambda i: (0, 0)),
            pl.BlockSpec((ED, 1), lambda i: (0, 0)),
        ],
        out_specs=[
            pl.BlockSpec((BE, ED), lambda i: (i, 0)),
            pl.BlockSpec((ED, 1, BE), lambda i: (0, 0, i)),
        ],
        out_shape=[
            jax.ShapeDtypeStruct((E, ED), jnp.float32),
            jax.ShapeDtypeStruct((ED, 1, E), jnp.float32),
        ],
    )(sg, qg, edges, w1e, b1, w2, b2, b2c)


def _node_body(x, p0, p1, w1a, w1b, b1, w2, b2, out):
    xv = x[...]
    aggt = p0[...] + p1[...]
    pre = (jnp.dot(xv, w1a[...], preferred_element_type=jnp.float32)
           + lax.dot_general(aggt, w1b[...], (((0,), (0,)), ((), ())),
                             preferred_element_type=jnp.float32)
           + b1[...])
    hn = jax.nn.gelu(pre)
    out[...] = xv + jnp.dot(hn, w2[...], preferred_element_type=jnp.float32) + b2[...]


def _tc_node(nodes, p0, p1, w1a, w1b, b1, w2, b2):
    return pl.pallas_call(
        _node_body,
        grid=(1,),
        in_specs=[
            pl.BlockSpec((N, ND), lambda i: (0, 0)),
            pl.BlockSpec((ED, N), lambda i: (0, 0)),
            pl.BlockSpec((ED, N), lambda i: (0, 0)),
            pl.BlockSpec((ND, H), lambda i: (0, 0)),
            pl.BlockSpec((ED, H), lambda i: (0, 0)),
            pl.BlockSpec((1, H), lambda i: (0, 0)),
            pl.BlockSpec((H, ND), lambda i: (0, 0)),
            pl.BlockSpec((1, ND), lambda i: (0, 0)),
        ],
        out_specs=pl.BlockSpec((N, ND), lambda i: (0, 0)),
        out_shape=jax.ShapeDtypeStruct((N, ND), jnp.float32),
    )(nodes, p0, p1, w1a, w1b, b1, w2, b2)


def kernel(nodes, edges, receivers, senders,
           edge_W1, edge_b1, edge_W2, edge_b2,
           node_W1, node_b1, node_W2, node_b2):
    w1s = edge_W1[:ND]
    w1r = edge_W1[ND:2 * ND]
    w1e = edge_W1[2 * ND:]
    p, q = _tc_prep(nodes, w1s, w1r)

    gather = _build_sc_gather()
    sg = gather(p, senders)
    qg = gather(q, receivers)

    edges_out, ne_t = _tc_edge(
        sg, qg, edges, w1e, edge_b1.reshape(1, H), edge_W2,
        edge_b2.reshape(1, ED), edge_b2.reshape(ED, 1))

    zeros_acc = jnp.zeros((N,), jnp.float32)
    parts = _build_sc_scatter()(receivers, ne_t, zeros_acc)

    nodes_out = _tc_node(
        nodes, parts[0, :, 0, :], parts[1, :, 0, :],
        node_W1[:ND], node_W1[ND:], node_b1.reshape(1, H),
        node_W2, node_b2.reshape(1, ND))
    return nodes_out, edges_out


# speedup vs baseline: 1.2485x; 1.2452x over previous
import functools

import jax
import jax.numpy as jnp
from jax import lax
from jax.experimental import pallas as pl
from jax.experimental.pallas import tpu as pltpu
from jax.experimental.pallas import tpu_sc as plsc

N = 10000
E = 320000
ND = 128
ED = 16
H = 128

NC = 2
NS = 16
NW = NC * NS
CHUNK = 80
GRP = 5
GEDGE = CHUNK * GRP
NGROUP = E // GEDGE
GPW = NGROUP // NW
NPAIR = GPW // 2
HD2 = H // 2

def _sc_mesh():
    return plsc.VectorSubcoreMesh(
        core_axis_name="c", subcore_axis_name="s", num_cores=NC, num_subcores=NS)


@functools.lru_cache(maxsize=None)
def _build_sc_gather():
    HL = HD2

    @functools.partial(
        pl.kernel,
        out_type=jax.ShapeDtypeStruct((E, HL), jnp.int32),
        mesh=_sc_mesh(),
        compiler_params=pltpu.CompilerParams(use_tc_tiling_on_sc=False,
                                             needs_layout_passes=False),
        scratch_types=[
            pltpu.VMEM((GEDGE,), jnp.int32),
            pltpu.VMEM((GEDGE,), jnp.int32),
            pltpu.VMEM((GEDGE,), jnp.int32),
            pltpu.VMEM((GEDGE,), jnp.int32),
            pltpu.VMEM((GEDGE, HL), jnp.int32),
            pltpu.VMEM((GEDGE, HL), jnp.int32),
            pltpu.VMEM((GEDGE, HL), jnp.int32),
            pltpu.VMEM((GEDGE, HL), jnp.int32),
            pltpu.SemaphoreType.DMA,
            pltpu.SemaphoreType.DMA,
            pltpu.SemaphoreType.DMA,
            pltpu.SemaphoreType.DMA,
        ],
    )
    def _sc_gather(p_hbm, q_hbm, sidx_hbm, ridx_hbm, out_hbm,
                   si0, ri0, si1, ri1, bp0, bq0, bp1, bq1,
                   semg0, semg1, semw0, semw1):
        cid = lax.axis_index("c")
        sid = lax.axis_index("s")
        wid = sid * NC + cid
        g0 = wid * GPW

        def stage(g, siv, riv, bp, bq, semg):
            pltpu.sync_copy(sidx_hbm.at[pl.ds(g * GEDGE, GEDGE)], siv)
            pltpu.sync_copy(ridx_hbm.at[pl.ds(g * GEDGE, GEDGE)], riv)
            for t in range(GRP):
                pltpu.async_copy(p_hbm.at[siv.at[pl.ds(t * CHUNK, CHUNK)]],
                                 bp.at[pl.ds(t * CHUNK, CHUNK), :], semg)
                pltpu.async_copy(q_hbm.at[riv.at[pl.ds(t * CHUNK, CHUNK)]],
                                 bq.at[pl.ds(t * CHUNK, CHUNK), :], semg)

        def drain_gather(bp, bq, semg):
            pltpu.make_async_copy(out_hbm.at[pl.ds(0, GEDGE), :], bp, semg).wait()
            pltpu.make_async_copy(out_hbm.at[pl.ds(0, GEDGE), :], bq, semg).wait()

        def add_packed(bp, bq):
            def row(r, carry):
                for l in range(HL // 16):
                    sl = pl.ds(l * 16, 16)
                    a = plsc.bitcast(bp[r, sl], jnp.bfloat16)
                    b = plsc.bitcast(bq[r, sl], jnp.bfloat16)
                    bp[r, sl] = plsc.bitcast(a + b, jnp.int32)
                return carry

            lax.fori_loop(0, GEDGE, row, 0)

        def fire_wb(g, bp, semw):
            pltpu.async_copy(bp, out_hbm.at[pl.ds(g * GEDGE, GEDGE), :], semw)

        def wait_wb(bp, semw):
            pltpu.make_async_copy(bp, out_hbm.at[pl.ds(0, GEDGE), :], semw).wait()

        def body(it, carry):
            ga = g0 + 2 * it
            gb = ga + 1

            @pl.when(it > 0)
            def _():
                wait_wb(bp0, semw0)

            stage(ga, si0, ri0, bp0, bq0, semg0)

            @pl.when(it > 0)
            def _():
                wait_wb(bp1, semw1)

            stage(gb, si1, ri1, bp1, bq1, semg1)
            drain_gather(bp0, bq0, semg0)
            add_packed(bp0, bq0)
            fire_wb(ga, bp0, semw0)
            drain_gather(bp1, bq1, semg1)
            add_packed(bp1, bq1)
            fire_wb(gb, bp1, semw1)
            return carry

        lax.fori_loop(0, NPAIR, body, 0)
        ge = g0 + 2 * NPAIR
        wait_wb(bp0, semw0)
        stage(ge, si0, ri0, bp0, bq0, semg0)
        drain_gather(bp0, bq0, semg0)
        add_packed(bp0, bq0)
        fire_wb(ge, bp0, semw0)
        wait_wb(bp0, semw0)
        wait_wb(bp1, semw1)

    return _sc_gather


E2 = E // NC
SCH = 6400
NSCH = E2 // SCH


@functools.lru_cache(maxsize=None)
def _build_sc_scatter():
    @functools.partial(
        pl.kernel,
        out_type=jax.ShapeDtypeStruct((NC, ED, N), jnp.float32),
        mesh=_sc_mesh(),
        compiler_params=pltpu.CompilerParams(needs_layout_passes=False,
                                             use_tc_tiling_on_sc=False),
        scratch_types=[
            pltpu.VMEM((SCH,), jnp.int32),
            pltpu.VMEM((SCH,), jnp.float32),
            pltpu.VMEM((SCH,), jnp.int32),
            pltpu.VMEM((SCH,), jnp.float32),
            pltpu.VMEM((N,), jnp.float32),
            pltpu.SemaphoreType.DMA,
            pltpu.SemaphoreType.DMA,
        ],
    )
    def _sc_scatter(ridx_hbm, net_hbm, zeros_hbm, out_hbm,
                    idx0, val0, idx1, val1, acc, sem0, sem1):
        cid = lax.axis_index("c")
        sid = lax.axis_index("s")
        base = cid * E2
        pltpu.sync_copy(zeros_hbm, acc)

        def fire(k, idxb, valb, sem):
            pltpu.async_copy(ridx_hbm.at[pl.ds(base + k * SCH, SCH)], idxb, sem)
            pltpu.async_copy(net_hbm.at[sid, pl.ds(base + k * SCH, SCH)], valb, sem)

        def drain(idxb, valb, sem):
            pltpu.make_async_copy(ridx_hbm.at[pl.ds(base, SCH)], idxb, sem).wait()
            pltpu.make_async_copy(net_hbm.at[sid, pl.ds(base, SCH)], valb, sem).wait()

        def compute(idxb, valb):
            def body(i, c2):
                for t in range(8):
                    iv = idxb[pl.ds((i * 8 + t) * 16, 16)]
                    vv = valb[pl.ds((i * 8 + t) * 16, 16)]
                    plsc.addupdate_scatter(acc, [iv], vv)
                return c2

            lax.fori_loop(0, SCH // 128, body, 0)

        fire(0, idx0, val0, sem0)

        def pair(it, carry):
            ka = 2 * it
            fire(ka + 1, idx1, val1, sem1)
            drain(idx0, val0, sem0)
            compute(idx0, val0)
            fire(ka + 2, idx0, val0, sem0)
            drain(idx1, val1, sem1)
            compute(idx1, val1)
            return carry

        lax.fori_loop(0, NSCH // 2, pair, 0)
        drain(idx0, val0, sem0)
        compute(idx0, val0)
        pltpu.sync_copy(acc, out_hbm.at[cid, sid])

    return _sc_scatter


def _pack_bf16_pair(lo, hi):
    def rn(x):
        b = lax.bitcast_convert_type(x, jnp.uint32)
        return (b + 0x7FFF + ((b >> 16) & 1)) >> 16

    word = rn(lo) | (rn(hi) << 16)
    return lax.bitcast_convert_type(word, jnp.int32)


def _unpack_bf16_pair(w):
    b = lax.bitcast_convert_type(w, jnp.uint32)
    lo = lax.bitcast_convert_type(b << 16, jnp.float32)
    hi = lax.bitcast_convert_type(b & jnp.uint32(0xFFFF0000), jnp.float32)
    return lo, hi


def _prep_body(x, w1s, w1r, p_out, q_out):
    xv = x[...]
    pf = jnp.dot(xv, w1s[...], preferred_element_type=jnp.float32)
    qf = jnp.dot(xv, w1r[...], preferred_element_type=jnp.float32)
    p_out[...] = _pack_bf16_pair(pf[:, :HD2], pf[:, HD2:])
    q_out[...] = _pack_bf16_pair(qf[:, :HD2], qf[:, HD2:])


def _tc_prep(nodes, w1s, w1r):
    BN = 2000
    grid = (N // BN,)
    return pl.pallas_call(
        _prep_body,
        grid=grid,
        in_specs=[
            pl.BlockSpec((BN, ND), lambda i: (i, 0)),
            pl.BlockSpec((ND, H), lambda i: (0, 0)),
            pl.BlockSpec((ND, H), lambda i: (0, 0)),
        ],
        out_specs=[
            pl.BlockSpec((BN, HD2), lambda i: (i, 0)),
            pl.BlockSpec((BN, HD2), lambda i: (i, 0)),
        ],
        out_shape=[
            jax.ShapeDtypeStruct((N, HD2), jnp.int32),
            jax.ShapeDtypeStruct((N, HD2), jnp.int32),
        ],
    )(nodes, w1s, w1r)


def _edge_body(g, e, w1e, b1, w2, b2, b2c, eo_out, net_out):
    ev = e[...]
    g_lo, g_hi = _unpack_bf16_pair(g[...])
    t = jnp.dot(ev, w1e[...], preferred_element_type=jnp.float32) + b1[...]
    w2v = w2[...]
    h_lo = jax.nn.gelu(g_lo + t[:, :HD2])
    h_hi = jax.nn.gelu(g_hi + t[:, HD2:])
    ne = (jnp.dot(h_lo, w2v[:HD2], preferred_element_type=jnp.float32)
          + jnp.dot(h_hi, w2v[HD2:], preferred_element_type=jnp.float32)
          + b2[...])
    eo_out[...] = ev + ne
    net = (lax.dot_general(w2v[:HD2], h_lo, (((0,), (1,)), ((), ())),
                           preferred_element_type=jnp.float32)
           + lax.dot_general(w2v[HD2:], h_hi, (((0,), (1,)), ((), ())),
                             preferred_element_type=jnp.float32))
    net_out[...] = net + b2c[...]


def _tc_edge(g, edges, w1e, b1, w2, b2, b2c):
    BE = 2560
    grid = (E // BE,)
    return pl.pallas_call(
        _edge_body,
        grid=grid,
        in_specs=[
            pl.BlockSpec((BE, HD2), lambda i: (i, 0)),
            pl.BlockSpec((BE, ED), lambda i: (i, 0)),
            pl.BlockSpec((ED, H), lambda i: (0, 0)),
            pl.BlockSpec((1, H), lambda i: (0, 0)),
            pl.BlockSpec((H, ED), lambda i: (0, 0)),
            pl.BlockSpec((1, ED), lambda i: (0, 0)),
            pl.BlockSpec((ED, 1), lambda i: (0, 0)),
        ],
        out_specs=[
            pl.BlockSpec((BE, ED), lambda i: (i, 0)),
            pl.BlockSpec((ED, BE), lambda i: (0, i)),
        ],
        out_shape=[
            jax.ShapeDtypeStruct((E, ED), jnp.float32),
            jax.ShapeDtypeStruct((ED, E), jnp.float32),
        ],
    )(g, edges, w1e, b1, w2, b2, b2c)


def _node_body(x, p0, p1, w1a, w1b, b1, w2, b2, out):
    xv = x[...]
    aggt = p0[...] + p1[...]
    pre = (jnp.dot(xv, w1a[...], preferred_element_type=jnp.float32)
           + lax.dot_general(aggt, w1b[...], (((0,), (0,)), ((), ())),
                             preferred_element_type=jnp.float32)
           + b1[...])
    hn = jax.nn.gelu(pre)
    out[...] = xv + jnp.dot(hn, w2[...], preferred_element_type=jnp.float32) + b2[...]


def _tc_node(nodes, p0, p1, w1a, w1b, b1, w2, b2):
    return pl.pallas_call(
        _node_body,
        grid=(1,),
        in_specs=[
            pl.BlockSpec((N, ND), lambda i: (0, 0)),
            pl.BlockSpec((ED, N), lambda i: (0, 0)),
            pl.BlockSpec((ED, N), lambda i: (0, 0)),
            pl.BlockSpec((ND, H), lambda i: (0, 0)),
            pl.BlockSpec((ED, H), lambda i: (0, 0)),
            pl.BlockSpec((1, H), lambda i: (0, 0)),
            pl.BlockSpec((H, ND), lambda i: (0, 0)),
            pl.BlockSpec((1, ND), lambda i: (0, 0)),
        ],
        out_specs=pl.BlockSpec((N, ND), lambda i: (0, 0)),
        out_shape=jax.ShapeDtypeStruct((N, ND), jnp.float32),
    )(nodes, p0, p1, w1a, w1b, b1, w2, b2)


def kernel(nodes, edges, receivers, senders,
           edge_W1, edge_b1, edge_W2, edge_b2,
           node_W1, node_b1, node_W2, node_b2):
    w1s = edge_W1[:ND]
    w1r = edge_W1[ND:2 * ND]
    w1e = edge_W1[2 * ND:]
    p, q = _tc_prep(nodes, w1s, w1r)

    g = _build_sc_gather()(p, q, senders, receivers)

    edges_out, ne_t = _tc_edge(
        g, edges, w1e, edge_b1.reshape(1, H), edge_W2,
        edge_b2.reshape(1, ED), edge_b2.reshape(ED, 1))

    zeros_acc = jnp.zeros((N,), jnp.float32)
    parts = _build_sc_scatter()(receivers, ne_t, zeros_acc)

    nodes_out = _tc_node(
        nodes, parts[0], parts[1],
        node_W1[:ND], node_W1[ND:], node_b1.reshape(1, H),
        node_W2, node_b2.reshape(1, ND))
    return nodes_out, edges_out


# speedup vs baseline: 1.2514x; 1.0023x over previous
import functools

import jax
import jax.numpy as jnp
from jax import lax
from jax.experimental import pallas as pl
from jax.experimental.pallas import tpu as pltpu
from jax.experimental.pallas import tpu_sc as plsc

N = 10000
E = 320000
ND = 128
ED = 16
H = 128

NC = 2
NS = 16
NW = NC * NS
CHUNK = 80
GRP = 5
GEDGE = CHUNK * GRP
NGROUP = E // GEDGE
GPW = NGROUP // NW
NPAIR = GPW // 2
HD2 = H // 2

def _sc_mesh():
    return plsc.VectorSubcoreMesh(
        core_axis_name="c", subcore_axis_name="s", num_cores=NC, num_subcores=NS)


@functools.lru_cache(maxsize=None)
def _build_sc_gather():
    HL = HD2

    @functools.partial(
        pl.kernel,
        out_type=jax.ShapeDtypeStruct((E, HL), jnp.int32),
        mesh=_sc_mesh(),
        compiler_params=pltpu.CompilerParams(use_tc_tiling_on_sc=False,
                                             needs_layout_passes=False),
        scratch_types=[
            pltpu.VMEM((GEDGE,), jnp.int32),
            pltpu.VMEM((GEDGE,), jnp.int32),
            pltpu.VMEM((GEDGE,), jnp.int32),
            pltpu.VMEM((GEDGE,), jnp.int32),
            pltpu.VMEM((GEDGE, HL), jnp.int32),
            pltpu.VMEM((GEDGE, HL), jnp.int32),
            pltpu.VMEM((GEDGE, HL), jnp.int32),
            pltpu.VMEM((GEDGE, HL), jnp.int32),
            pltpu.SemaphoreType.DMA,
            pltpu.SemaphoreType.DMA,
            pltpu.SemaphoreType.DMA,
            pltpu.SemaphoreType.DMA,
        ],
    )
    def _sc_gather(p_hbm, q_hbm, sidx_hbm, ridx_hbm, out_hbm,
                   si0, ri0, si1, ri1, bp0, bq0, bp1, bq1,
                   semg0, semg1, semw0, semw1):
        cid = lax.axis_index("c")
        sid = lax.axis_index("s")
        wid = sid * NC + cid
        g0 = wid * GPW

        def stage(g, siv, riv, bp, bq, semg):
            pltpu.sync_copy(sidx_hbm.at[pl.ds(g * GEDGE, GEDGE)], siv)
            pltpu.sync_copy(ridx_hbm.at[pl.ds(g * GEDGE, GEDGE)], riv)
            for t in range(GRP):
                pltpu.async_copy(p_hbm.at[siv.at[pl.ds(t * CHUNK, CHUNK)]],
                                 bp.at[pl.ds(t * CHUNK, CHUNK), :], semg)
                pltpu.async_copy(q_hbm.at[riv.at[pl.ds(t * CHUNK, CHUNK)]],
                                 bq.at[pl.ds(t * CHUNK, CHUNK), :], semg)

        def drain_gather(bp, bq, semg):
            pltpu.make_async_copy(out_hbm.at[pl.ds(0, GEDGE), :], bp, semg).wait()
            pltpu.make_async_copy(out_hbm.at[pl.ds(0, GEDGE), :], bq, semg).wait()

        def add_packed(bp, bq):
            def row(r2, carry):
                for u in range(2):
                    r = r2 * 2 + u
                    for l in range(HL // 16):
                        sl = pl.ds(l * 16, 16)
                        a = plsc.bitcast(bp[r, sl], jnp.bfloat16)
                        b = plsc.bitcast(bq[r, sl], jnp.bfloat16)
                        bp[r, sl] = plsc.bitcast(a + b, jnp.int32)
                return carry

            lax.fori_loop(0, GEDGE // 2, row, 0)

        def fire_wb(g, bp, semw):
            pltpu.async_copy(bp, out_hbm.at[pl.ds(g * GEDGE, GEDGE), :], semw)

        def wait_wb(bp, semw):
            pltpu.make_async_copy(bp, out_hbm.at[pl.ds(0, GEDGE), :], semw).wait()

        def body(it, carry):
            ga = g0 + 2 * it
            gb = ga + 1

            @pl.when(it > 0)
            def _():
                wait_wb(bp0, semw0)

            stage(ga, si0, ri0, bp0, bq0, semg0)

            @pl.when(it > 0)
            def _():
                wait_wb(bp1, semw1)

            stage(gb, si1, ri1, bp1, bq1, semg1)
            drain_gather(bp0, bq0, semg0)
            add_packed(bp0, bq0)
            fire_wb(ga, bp0, semw0)
            drain_gather(bp1, bq1, semg1)
            add_packed(bp1, bq1)
            fire_wb(gb, bp1, semw1)
            return carry

        lax.fori_loop(0, NPAIR, body, 0)
        ge = g0 + 2 * NPAIR
        wait_wb(bp0, semw0)
        stage(ge, si0, ri0, bp0, bq0, semg0)
        drain_gather(bp0, bq0, semg0)
        add_packed(bp0, bq0)
        fire_wb(ge, bp0, semw0)
        wait_wb(bp0, semw0)
        wait_wb(bp1, semw1)

    return _sc_gather


E2 = E // NC
SCH = 6400
NSCH = E2 // SCH


@functools.lru_cache(maxsize=None)
def _build_sc_scatter():
    @functools.partial(
        pl.kernel,
        out_type=jax.ShapeDtypeStruct((NC, ED, N), jnp.float32),
        mesh=_sc_mesh(),
        compiler_params=pltpu.CompilerParams(needs_layout_passes=False,
                                             use_tc_tiling_on_sc=False),
        scratch_types=[
            pltpu.VMEM((SCH,), jnp.int32),
            pltpu.VMEM((SCH,), jnp.float32),
            pltpu.VMEM((SCH,), jnp.int32),
            pltpu.VMEM((SCH,), jnp.float32),
            pltpu.VMEM((N,), jnp.float32),
            pltpu.SemaphoreType.DMA,
            pltpu.SemaphoreType.DMA,
        ],
    )
    def _sc_scatter(ridx_hbm, net_hbm, zeros_hbm, out_hbm,
                    idx0, val0, idx1, val1, acc, sem0, sem1):
        cid = lax.axis_index("c")
        sid = lax.axis_index("s")
        base = cid * E2
        pltpu.sync_copy(zeros_hbm, acc)

        def fire(k, idxb, valb, sem):
            pltpu.async_copy(ridx_hbm.at[pl.ds(base + k * SCH, SCH)], idxb, sem)
            pltpu.async_copy(net_hbm.at[sid, pl.ds(base + k * SCH, SCH)], valb, sem)

        def drain(idxb, valb, sem):
            pltpu.make_async_copy(ridx_hbm.at[pl.ds(base, SCH)], idxb, sem).wait()
            pltpu.make_async_copy(net_hbm.at[sid, pl.ds(base, SCH)], valb, sem).wait()

        def compute(idxb, valb):
            def body(i, c2):
                for t in range(8):
                    iv = idxb[pl.ds((i * 8 + t) * 16, 16)]
                    vv = valb[pl.ds((i * 8 + t) * 16, 16)]
                    plsc.addupdate_scatter(acc, [iv], vv)
                return c2

            lax.fori_loop(0, SCH // 128, body, 0)

        fire(0, idx0, val0, sem0)

        def pair(it, carry):
            ka = 2 * it
            fire(ka + 1, idx1, val1, sem1)
            drain(idx0, val0, sem0)
            compute(idx0, val0)
            fire(ka + 2, idx0, val0, sem0)
            drain(idx1, val1, sem1)
            compute(idx1, val1)
            return carry

        lax.fori_loop(0, NSCH // 2, pair, 0)
        drain(idx0, val0, sem0)
        compute(idx0, val0)
        pltpu.sync_copy(acc, out_hbm.at[cid, sid])

    return _sc_scatter


def _pack_bf16_pair(lo, hi):
    def rn(x):
        b = lax.bitcast_convert_type(x, jnp.uint32)
        return (b + 0x7FFF + ((b >> 16) & 1)) >> 16

    word = rn(lo) | (rn(hi) << 16)
    return lax.bitcast_convert_type(word, jnp.int32)


def _unpack_bf16_pair(w):
    b = lax.bitcast_convert_type(w, jnp.uint32)
    lo = lax.bitcast_convert_type(b << 16, jnp.float32)
    hi = lax.bitcast_convert_type(b & jnp.uint32(0xFFFF0000), jnp.float32)
    return lo, hi


def _prep_body(x, w1s, w1r, p_out, q_out):
    xv = x[...]
    pf = jnp.dot(xv, w1s[...], preferred_element_type=jnp.float32)
    qf = jnp.dot(xv, w1r[...], preferred_element_type=jnp.float32)
    p_out[...] = _pack_bf16_pair(pf[:, :HD2], pf[:, HD2:])
    q_out[...] = _pack_bf16_pair(qf[:, :HD2], qf[:, HD2:])


def _tc_prep(nodes, w1s, w1r):
    BN = 2000
    grid = (N // BN,)
    return pl.pallas_call(
        _prep_body,
        grid=grid,
        in_specs=[
            pl.BlockSpec((BN, ND), lambda i: (i, 0)),
            pl.BlockSpec((ND, H), lambda i: (0, 0)),
            pl.BlockSpec((ND, H), lambda i: (0, 0)),
        ],
        out_specs=[
            pl.BlockSpec((BN, HD2), lambda i: (i, 0)),
            pl.BlockSpec((BN, HD2), lambda i: (i, 0)),
        ],
        out_shape=[
            jax.ShapeDtypeStruct((N, HD2), jnp.int32),
            jax.ShapeDtypeStruct((N, HD2), jnp.int32),
        ],
    )(nodes, w1s, w1r)


def _edge_body(g, e, w1e, b1, w2, b2, eo_out, net_out):
    ev = e[...]
    g_lo, g_hi = _unpack_bf16_pair(g[...])
    t = jnp.dot(ev, w1e[...], preferred_element_type=jnp.float32) + b1[...]
    w2v = w2[...]
    h_lo = jax.nn.gelu(g_lo + t[:, :HD2])
    h_hi = jax.nn.gelu(g_hi + t[:, HD2:])
    ne = (jnp.dot(h_lo, w2v[:HD2], preferred_element_type=jnp.float32)
          + jnp.dot(h_hi, w2v[HD2:], preferred_element_type=jnp.float32)
          + b2[...])
    eo_out[...] = ev + ne
    net_out[...] = ne.T


def _tc_edge(g, edges, w1e, b1, w2, b2):
    BE = 2560
    grid = (E // BE,)
    return pl.pallas_call(
        _edge_body,
        grid=grid,
        in_specs=[
            pl.BlockSpec((BE, HD2), lambda i: (i, 0)),
            pl.BlockSpec((BE, ED), lambda i: (i, 0)),
            pl.BlockSpec((ED, H), lambda i: (0, 0)),
            pl.BlockSpec((1, H), lambda i: (0, 0)),
            pl.BlockSpec((H, ED), lambda i: (0, 0)),
            pl.BlockSpec((1, ED), lambda i: (0, 0)),
        ],
        out_specs=[
            pl.BlockSpec((BE, ED), lambda i: (i, 0)),
            pl.BlockSpec((ED, BE), lambda i: (0, i)),
        ],
        out_shape=[
            jax.ShapeDtypeStruct((E, ED), jnp.float32),
            jax.ShapeDtypeStruct((ED, E), jnp.float32),
        ],
    )(g, edges, w1e, b1, w2, b2)


def _node_body(x, p0, p1, w1a, w1b, b1, w2, b2, out):
    xv = x[...]
    aggt = p0[...] + p1[...]
    pre = (jnp.dot(xv, w1a[...], preferred_element_type=jnp.float32)
           + lax.dot_general(aggt, w1b[...], (((0,), (0,)), ((), ())),
                             preferred_element_type=jnp.float32)
           + b1[...])
    hn = jax.nn.gelu(pre)
    out[...] = xv + jnp.dot(hn, w2[...], preferred_element_type=jnp.float32) + b2[...]


def _tc_node(nodes, p0, p1, w1a, w1b, b1, w2, b2):
    return pl.pallas_call(
        _node_body,
        grid=(1,),
        in_specs=[
            pl.BlockSpec((N, ND), lambda i: (0, 0)),
            pl.BlockSpec((ED, N), lambda i: (0, 0)),
            pl.BlockSpec((ED, N), lambda i: (0, 0)),
            pl.BlockSpec((ND, H), lambda i: (0, 0)),
            pl.BlockSpec((ED, H), lambda i: (0, 0)),
            pl.BlockSpec((1, H), lambda i: (0, 0)),
            pl.BlockSpec((H, ND), lambda i: (0, 0)),
            pl.BlockSpec((1, ND), lambda i: (0, 0)),
        ],
        out_specs=pl.BlockSpec((N, ND), lambda i: (0, 0)),
        out_shape=jax.ShapeDtypeStruct((N, ND), jnp.float32),
    )(nodes, p0, p1, w1a, w1b, b1, w2, b2)


def kernel(nodes, edges, receivers, senders,
           edge_W1, edge_b1, edge_W2, edge_b2,
           node_W1, node_b1, node_W2, node_b2):
    w1s = edge_W1[:ND]
    w1r = edge_W1[ND:2 * ND]
    w1e = edge_W1[2 * ND:]
    p, q = _tc_prep(nodes, w1s, w1r)

    g = _build_sc_gather()(p, q, senders, receivers)

    edges_out, ne_t = _tc_edge(
        g, edges, w1e, edge_b1.reshape(1, H), edge_W2, edge_b2.reshape(1, ED))

    zeros_acc = jnp.zeros((N,), jnp.float32)
    parts = _build_sc_scatter()(receivers, ne_t, zeros_acc)

    nodes_out = _tc_node(
        nodes, parts[0], parts[1],
        node_W1[:ND], node_W1[ND:], node_b1.reshape(1, H),
        node_W2, node_b2.reshape(1, ND))
    return nodes_out, edges_out


# speedup vs baseline: 1.2922x; 1.0326x over previous
import functools

import jax
import jax.numpy as jnp
from jax import lax
from jax.experimental import pallas as pl
from jax.experimental.pallas import tpu as pltpu
from jax.experimental.pallas import tpu_sc as plsc

N = 10000
E = 320000
ND = 128
ED = 16
H = 128

NC = 2
NS = 16
NW = NC * NS
CHUNK = 80
GRP = 5
GEDGE = CHUNK * GRP
NGROUP = E // GEDGE
GPW = NGROUP // NW
NPAIR = GPW // 2
HD2 = H // 2

def _sc_mesh():
    return plsc.VectorSubcoreMesh(
        core_axis_name="c", subcore_axis_name="s", num_cores=NC, num_subcores=NS)


@functools.lru_cache(maxsize=None)
def _build_sc_gather():
    HL = HD2

    @functools.partial(
        pl.kernel,
        out_type=jax.ShapeDtypeStruct((E, HL), jnp.int32),
        mesh=_sc_mesh(),
        compiler_params=pltpu.CompilerParams(use_tc_tiling_on_sc=False,
                                             needs_layout_passes=False),
        scratch_types=[
            pltpu.VMEM((GEDGE,), jnp.int32),
            pltpu.VMEM((GEDGE,), jnp.int32),
            pltpu.VMEM((GEDGE,), jnp.int32),
            pltpu.VMEM((GEDGE,), jnp.int32),
            pltpu.VMEM((GEDGE, HL), jnp.int32),
            pltpu.VMEM((GEDGE, HL), jnp.int32),
            pltpu.VMEM((GEDGE, HL), jnp.int32),
            pltpu.VMEM((GEDGE, HL), jnp.int32),
            pltpu.SemaphoreType.DMA,
            pltpu.SemaphoreType.DMA,
            pltpu.SemaphoreType.DMA,
            pltpu.SemaphoreType.DMA,
        ],
    )
    def _sc_gather(p_hbm, q_hbm, sidx_hbm, ridx_hbm, out_hbm,
                   si0, ri0, si1, ri1, bp0, bq0, bp1, bq1,
                   semg0, semg1, semw0, semw1):
        cid = lax.axis_index("c")
        sid = lax.axis_index("s")
        wid = sid * NC + cid
        g0 = wid * GPW

        def stage(g, siv, riv, bp, bq, semg):
            pltpu.sync_copy(sidx_hbm.at[pl.ds(g * GEDGE, GEDGE)], siv)
            pltpu.sync_copy(ridx_hbm.at[pl.ds(g * GEDGE, GEDGE)], riv)
            for t in range(GRP):
                pltpu.async_copy(p_hbm.at[siv.at[pl.ds(t * CHUNK, CHUNK)]],
                                 bp.at[pl.ds(t * CHUNK, CHUNK), :], semg)
                pltpu.async_copy(q_hbm.at[riv.at[pl.ds(t * CHUNK, CHUNK)]],
                                 bq.at[pl.ds(t * CHUNK, CHUNK), :], semg)

        def drain_gather(bp, bq, semg):
            pltpu.make_async_copy(out_hbm.at[pl.ds(0, GEDGE), :], bp, semg).wait()
            pltpu.make_async_copy(out_hbm.at[pl.ds(0, GEDGE), :], bq, semg).wait()

        def add_packed(bp, bq):
            def row(r2, carry):
                for u in range(2):
                    r = r2 * 2 + u
                    for l in range(HL // 16):
                        sl = pl.ds(l * 16, 16)
                        a = plsc.bitcast(bp[r, sl], jnp.bfloat16)
                        b = plsc.bitcast(bq[r, sl], jnp.bfloat16)
                        bp[r, sl] = plsc.bitcast(a + b, jnp.int32)
                return carry

            lax.fori_loop(0, GEDGE // 2, row, 0)

        def fire_wb(g, bp, semw):
            pltpu.async_copy(bp, out_hbm.at[pl.ds(g * GEDGE, GEDGE), :], semw)

        def wait_wb(bp, semw):
            pltpu.make_async_copy(bp, out_hbm.at[pl.ds(0, GEDGE), :], semw).wait()

        def body(it, carry):
            ga = g0 + 2 * it
            gb = ga + 1

            @pl.when(it > 0)
            def _():
                wait_wb(bp0, semw0)

            stage(ga, si0, ri0, bp0, bq0, semg0)

            @pl.when(it > 0)
            def _():
                wait_wb(bp1, semw1)

            stage(gb, si1, ri1, bp1, bq1, semg1)
            drain_gather(bp0, bq0, semg0)
            add_packed(bp0, bq0)
            fire_wb(ga, bp0, semw0)
            drain_gather(bp1, bq1, semg1)
            add_packed(bp1, bq1)
            fire_wb(gb, bp1, semw1)
            return carry

        lax.fori_loop(0, NPAIR, body, 0)
        ge = g0 + 2 * NPAIR
        wait_wb(bp0, semw0)
        stage(ge, si0, ri0, bp0, bq0, semg0)
        drain_gather(bp0, bq0, semg0)
        add_packed(bp0, bq0)
        fire_wb(ge, bp0, semw0)
        wait_wb(bp0, semw0)
        wait_wb(bp1, semw1)

    return _sc_gather


E2 = E // NC
SCH = 6400
NSCH = E2 // SCH


@functools.lru_cache(maxsize=None)
def _build_sc_scatter():
    @functools.partial(
        pl.kernel,
        out_type=jax.ShapeDtypeStruct((NC, ED, N), jnp.float32),
        mesh=_sc_mesh(),
        compiler_params=pltpu.CompilerParams(needs_layout_passes=False,
                                             use_tc_tiling_on_sc=False),
        scratch_types=[
            pltpu.VMEM((SCH,), jnp.int32),
            pltpu.VMEM((SCH,), jnp.float32),
            pltpu.VMEM((SCH,), jnp.int32),
            pltpu.VMEM((SCH,), jnp.float32),
            pltpu.VMEM((N,), jnp.float32),
            pltpu.SemaphoreType.DMA,
            pltpu.SemaphoreType.DMA,
        ],
    )
    def _sc_scatter(ridx_hbm, net_hbm, zeros_hbm, out_hbm,
                    idx0, val0, idx1, val1, acc, sem0, sem1):
        cid = lax.axis_index("c")
        sid = lax.axis_index("s")
        base = cid * E2
        pltpu.sync_copy(zeros_hbm, acc)

        def fire(k, idxb, valb, sem):
            pltpu.async_copy(ridx_hbm.at[pl.ds(base + k * SCH, SCH)], idxb, sem)
            pltpu.async_copy(net_hbm.at[sid, pl.ds(base + k * SCH, SCH)], valb, sem)

        def drain(idxb, valb, sem):
            pltpu.make_async_copy(ridx_hbm.at[pl.ds(base, SCH)], idxb, sem).wait()
            pltpu.make_async_copy(net_hbm.at[sid, pl.ds(base, SCH)], valb, sem).wait()

        def compute(idxb, valb):
            def body(i, c2):
                for t in range(8):
                    iv = idxb[pl.ds((i * 8 + t) * 16, 16)]
                    vv = valb[pl.ds((i * 8 + t) * 16, 16)]
                    plsc.addupdate_scatter(acc, [iv], vv)
                return c2

            lax.fori_loop(0, SCH // 128, body, 0)

        fire(0, idx0, val0, sem0)

        def pair(it, carry):
            ka = 2 * it
            fire(ka + 1, idx1, val1, sem1)
            drain(idx0, val0, sem0)
            compute(idx0, val0)
            fire(ka + 2, idx0, val0, sem0)
            drain(idx1, val1, sem1)
            compute(idx1, val1)
            return carry

        lax.fori_loop(0, NSCH // 2, pair, 0)
        drain(idx0, val0, sem0)
        compute(idx0, val0)
        pltpu.sync_copy(acc, out_hbm.at[cid, sid])

    return _sc_scatter


def _pack_bf16_pair(lo, hi):
    def rn(x):
        b = lax.bitcast_convert_type(x, jnp.uint32)
        return (b + 0x7FFF + ((b >> 16) & 1)) >> 16

    word = rn(lo) | (rn(hi) << 16)
    return lax.bitcast_convert_type(word, jnp.int32)


def _unpack_bf16_pair(w):
    b = lax.bitcast_convert_type(w, jnp.uint32)
    lo = lax.bitcast_convert_type(b << 16, jnp.float32)
    hi = lax.bitcast_convert_type(b & jnp.uint32(0xFFFF0000), jnp.float32)
    return lo, hi


def _prep_body(x, w1s, w1r, p_out, q_out):
    xv = x[...]
    pf = jnp.dot(xv, w1s[...], preferred_element_type=jnp.float32)
    qf = jnp.dot(xv, w1r[...], preferred_element_type=jnp.float32)
    p_out[...] = _pack_bf16_pair(pf[:, :HD2], pf[:, HD2:])
    q_out[...] = _pack_bf16_pair(qf[:, :HD2], qf[:, HD2:])


def _tc_prep(nodes, w1s, w1r):
    BN = 2000
    grid = (N // BN,)
    return pl.pallas_call(
        _prep_body,
        grid=grid,
        in_specs=[
            pl.BlockSpec((BN, ND), lambda i: (i, 0)),
            pl.BlockSpec((ND, H), lambda i: (0, 0)),
            pl.BlockSpec((ND, H), lambda i: (0, 0)),
        ],
        out_specs=[
            pl.BlockSpec((BN, HD2), lambda i: (i, 0)),
            pl.BlockSpec((BN, HD2), lambda i: (i, 0)),
        ],
        out_shape=[
            jax.ShapeDtypeStruct((N, HD2), jnp.int32),
            jax.ShapeDtypeStruct((N, HD2), jnp.int32),
        ],
    )(nodes, w1s, w1r)


def _edge_body(g, e, w1e, b1, w2, b2, eo_out, net_out):
    ev = e[...]
    g_lo, g_hi = _unpack_bf16_pair(g[...])
    t = jnp.dot(ev, w1e[...], preferred_element_type=jnp.float32) + b1[...]
    w2v = w2[...].astype(jnp.bfloat16)
    h_lo = jax.nn.gelu((g_lo + t[:, :HD2]).astype(jnp.bfloat16))
    h_hi = jax.nn.gelu((g_hi + t[:, HD2:]).astype(jnp.bfloat16))
    ne = (jnp.dot(h_lo, w2v[:HD2], preferred_element_type=jnp.float32)
          + jnp.dot(h_hi, w2v[HD2:], preferred_element_type=jnp.float32)
          + b2[...])
    eo_out[...] = ev + ne
    net_out[...] = ne.T


def _tc_edge(g, edges, w1e, b1, w2, b2):
    BE = 2560
    grid = (E // BE,)
    return pl.pallas_call(
        _edge_body,
        grid=grid,
        in_specs=[
            pl.BlockSpec((BE, HD2), lambda i: (i, 0)),
            pl.BlockSpec((BE, ED), lambda i: (i, 0)),
            pl.BlockSpec((ED, H), lambda i: (0, 0)),
            pl.BlockSpec((1, H), lambda i: (0, 0)),
            pl.BlockSpec((H, ED), lambda i: (0, 0)),
            pl.BlockSpec((1, ED), lambda i: (0, 0)),
        ],
        out_specs=[
            pl.BlockSpec((BE, ED), lambda i: (i, 0)),
            pl.BlockSpec((ED, BE), lambda i: (0, i)),
        ],
        out_shape=[
            jax.ShapeDtypeStruct((E, ED), jnp.float32),
            jax.ShapeDtypeStruct((ED, E), jnp.float32),
        ],
    )(g, edges, w1e, b1, w2, b2)


def _node_body(x, p0, p1, w1a, w1b, b1, w2, b2, out):
    xv = x[...]
    aggt = p0[...] + p1[...]
    pre = (jnp.dot(xv, w1a[...], preferred_element_type=jnp.float32)
           + lax.dot_general(aggt, w1b[...], (((0,), (0,)), ((), ())),
                             preferred_element_type=jnp.float32)
           + b1[...])
    hn = jax.nn.gelu(pre)
    out[...] = xv + jnp.dot(hn, w2[...], preferred_element_type=jnp.float32) + b2[...]


def _tc_node(nodes, p0, p1, w1a, w1b, b1, w2, b2):
    return pl.pallas_call(
        _node_body,
        grid=(1,),
        in_specs=[
            pl.BlockSpec((N, ND), lambda i: (0, 0)),
            pl.BlockSpec((ED, N), lambda i: (0, 0)),
            pl.BlockSpec((ED, N), lambda i: (0, 0)),
            pl.BlockSpec((ND, H), lambda i: (0, 0)),
            pl.BlockSpec((ED, H), lambda i: (0, 0)),
            pl.BlockSpec((1, H), lambda i: (0, 0)),
            pl.BlockSpec((H, ND), lambda i: (0, 0)),
            pl.BlockSpec((1, ND), lambda i: (0, 0)),
        ],
        out_specs=pl.BlockSpec((N, ND), lambda i: (0, 0)),
        out_shape=jax.ShapeDtypeStruct((N, ND), jnp.float32),
    )(nodes, p0, p1, w1a, w1b, b1, w2, b2)


def kernel(nodes, edges, receivers, senders,
           edge_W1, edge_b1, edge_W2, edge_b2,
           node_W1, node_b1, node_W2, node_b2):
    w1s = edge_W1[:ND]
    w1r = edge_W1[ND:2 * ND]
    w1e = edge_W1[2 * ND:]
    p, q = _tc_prep(nodes, w1s, w1r)

    g = _build_sc_gather()(p, q, senders, receivers)

    edges_out, ne_t = _tc_edge(
        g, edges, w1e, edge_b1.reshape(1, H), edge_W2, edge_b2.reshape(1, ED))

    zeros_acc = jnp.zeros((N,), jnp.float32)
    parts = _build_sc_scatter()(receivers, ne_t, zeros_acc)

    nodes_out = _tc_node(
        nodes, parts[0], parts[1],
        node_W1[:ND], node_W1[ND:], node_b1.reshape(1, H),
        node_W2, node_b2.reshape(1, ND))
    return nodes_out, edges_out
